# Initial kernel scaffold; baseline (speedup 1.0000x reference)
import jax
import jax.numpy as jnp
from jax.experimental import pallas as pl


def kernel(low_context_vector, original_node, mask, low_init_w, W_ctx, b_ctx, W_vw, b_vw, Wq, bq, Wref, bref, v_ptr):
    raise NotImplementedError("write your pallas kernel here")


# speedup vs baseline: 1.7095x; 1.7095x over previous
import jax
import jax.numpy as jnp
from jax.experimental import pallas as pl

_C = 10.0
_NEG = -1e8
_EPS = 1e-10


def _decode(lcv_ref, g_ref, mask_ref, hmean_ref, liw_ref, wctx_ref, bctx_ref,
            wvw_ref, bvw_ref, wq_ref, bq_ref, wref_ref, bref_ref, vp_ref,
            idx_out_ref, lp_out_ref):
    lcv = lcv_ref[...]
    B, S, E = lcv.shape
    n = S - 1
    H = wq_ref.shape[1]

    h_bar = jnp.dot(hmean_ref[...], wctx_ref[...]) + bctx_ref[...]
    h_rest0 = jnp.dot(liw_ref[...], wvw_ref[...]) + bvw_ref[...]
    query0 = h_bar + h_rest0
    refm = (jnp.dot(lcv.reshape(B * S, E), wref_ref[...])
            + bref_ref[...]).reshape(B, S, H)
    vp = vp_ref[...]

    col = jax.lax.broadcasted_iota(jnp.int32, (B, S), 1)
    lane = jax.lax.broadcasted_iota(jnp.int32, (B, n), 1)
    msk0 = jnp.where(col == 0, 1.0, mask_ref[...])

    wq = wq_ref[...]
    bq = bq_ref[...]
    wvw = wvw_ref[...]
    bvw = bvw_ref[...]

    def body(i, carry):
        query, msk, init_h, idx_all, lp_all = carry
        q = jnp.dot(query, wq) + bq
        t = jnp.tanh(refm + q[:, None, :])
        s = jax.lax.dot_general(t.reshape(B * S, H), vp,
                                (((1,), (0,)), ((), ()))).reshape(B, S)
        u = _C * jnp.tanh(s)
        u = jnp.where(msk > 0, _NEG, u)
        m = jnp.max(u, axis=1, keepdims=True)
        e = jnp.exp(u - m)
        z = jnp.sum(e, axis=1, keepdims=True)
        p = e / z
        logits = jnp.log(p + _EPS)
        score = g_ref[i] + logits
        maxv = jnp.max(score, axis=1, keepdims=True)
        cand = jnp.where(score == maxv, col, S)
        idxc = jnp.min(cand, axis=1, keepdims=True)
        ohb = col == idxc
        lpc = jnp.max(jnp.where(ohb, logits, -jnp.inf), axis=1, keepdims=True)
        msk = jnp.where(ohb, 1.0, msk)
        h = jnp.sum(lcv * ohb.astype(jnp.float32)[:, :, None], axis=1)
        init_h = jnp.where(i == 0, h, init_h)
        cc = jnp.concatenate([init_h, h], axis=-1)
        query = h_bar + (jnp.dot(cc, wvw) + bvw)
        idx_all = jnp.where(lane == i, idxc, idx_all)
        lp_all = jnp.where(lane == i, lpc, lp_all)
        return query, msk, init_h, idx_all, lp_all

    init = (query0, msk0, jnp.zeros_like(h_bar),
            jnp.zeros((B, n), jnp.int32), jnp.zeros((B, n), jnp.float32))
    _, _, _, idx_all, lp_all = jax.lax.fori_loop(0, n, body, init)
    idx_out_ref[...] = idx_all
    lp_out_ref[...] = lp_all


def kernel(low_context_vector, original_node, mask, low_init_w, W_ctx, b_ctx,
           W_vw, b_vw, Wq, bq, Wref, bref, v_ptr):
    del original_node
    lcv = low_context_vector
    B, S, E = lcv.shape
    n = S - 1

    k = jax.random.key(42)
    gs = []
    for _ in range(n):
        k, sub = jax.random.split(k)
        gs.append(jax.random.gumbel(sub, (B, S), jnp.float32))
    g = jnp.stack(gs)

    h_mean = jax.lax.dot_general(
        jnp.ones((S,), jnp.float32), lcv, (((0,), (1,)), ((), ())),
        precision="highest") / S

    idxs, lps = pl.pallas_call(
        _decode,
        out_shape=(jax.ShapeDtypeStruct((B, n), jnp.int32),
                   jax.ShapeDtypeStruct((B, n), jnp.float32)),
    )(lcv, g, mask, h_mean, low_init_w.reshape(1, -1), W_ctx,
      b_ctx.reshape(1, -1), W_vw, b_vw.reshape(1, -1), Wq, bq.reshape(1, -1),
      Wref, bref.reshape(1, -1), v_ptr.reshape(-1, 1))
    return idxs, lps


# speedup vs baseline: 2.2336x; 1.3066x over previous
import jax
import jax.numpy as jnp
from jax.experimental import pallas as pl

_C = 10.0
_NEG = -1e8
_EPS = 1e-10


def _decode(lcv_ref, lcvw_ref, g_ref, mask_ref, hmean_ref, liw_ref, wctx_ref,
            bctx_ref, wvw_ref, bvw_ref, wq_ref, bq_ref, wref_ref, bref_ref,
            vblk_ref, idx_out_ref, lp_out_ref):
    lcv = lcv_ref[...]
    B, S, E = lcv.shape
    n = S - 1
    H = wq_ref.shape[1]

    h_bar = jnp.dot(hmean_ref[...], wctx_ref[...]) + bctx_ref[...]
    h_rest0 = jnp.dot(liw_ref[...], wvw_ref[...]) + bvw_ref[...]
    query0 = h_bar + h_rest0
    lcvw = lcvw_ref[...]
    wref = wref_ref[...]
    bref = bref_ref[...]
    refw = jnp.concatenate(
        [jnp.dot(lcvw[:, s * E:(s + 1) * E], wref) + bref for s in range(S)],
        axis=1)
    vblk = vblk_ref[...]

    col = jax.lax.broadcasted_iota(jnp.int32, (B, S), 1)
    lane = jax.lax.broadcasted_iota(jnp.int32, (B, n), 1)
    msk0 = jnp.where(col == 0, 1.0, mask_ref[...])

    wq = wq_ref[...]
    bq = bq_ref[...]
    wvw = wvw_ref[...]
    bvw = bvw_ref[...]

    def body(i, carry):
        query, msk, init_h, idx_all, lp_all = carry
        q = jnp.dot(query, wq) + bq
        qw = jnp.concatenate([q] * S, axis=1)
        t = jnp.tanh(refw + qw)
        s = jax.lax.dot_general(t, vblk, (((1,), (0,)), ((), ())))
        u = _C * jnp.tanh(s)
        u = jnp.where(msk > 0, _NEG, u)
        m = jnp.max(u, axis=1, keepdims=True)
        e = jnp.exp(u - m)
        z = jnp.sum(e, axis=1, keepdims=True)
        p = e / z
        logits = jnp.log(p + _EPS)
        score = g_ref[i] + logits
        maxv = jnp.max(score, axis=1, keepdims=True)
        cand = jnp.where(score == maxv, col, S)
        idxc = jnp.min(cand, axis=1, keepdims=True)
        ohb = col == idxc
        lpc = jnp.max(jnp.where(ohb, logits, -jnp.inf), axis=1, keepdims=True)
        msk = jnp.where(ohb, 1.0, msk)
        h = jnp.sum(lcv * ohb.astype(jnp.float32)[:, :, None], axis=1)
        init_h = jnp.where(i == 0, h, init_h)
        cc = jnp.concatenate([init_h, h], axis=-1)
        query = h_bar + (jnp.dot(cc, wvw) + bvw)
        idx_all = jnp.where(lane == i, idxc, idx_all)
        lp_all = jnp.where(lane == i, lpc, lp_all)
        return query, msk, init_h, idx_all, lp_all

    init = (query0, msk0, jnp.zeros_like(h_bar),
            jnp.zeros((B, n), jnp.int32), jnp.zeros((B, n), jnp.float32))
    _, _, _, idx_all, lp_all = jax.lax.fori_loop(0, n, body, init)
    idx_out_ref[...] = idx_all
    lp_out_ref[...] = lp_all


def kernel(low_context_vector, original_node, mask, low_init_w, W_ctx, b_ctx,
           W_vw, b_vw, Wq, bq, Wref, bref, v_ptr):
    del original_node
    lcv = low_context_vector
    B, S, E = lcv.shape
    H = Wq.shape[1]
    n = S - 1

    k = jax.random.key(42)
    gs = []
    for _ in range(n):
        k, sub = jax.random.split(k)
        gs.append(jax.random.gumbel(sub, (B, S), jnp.float32))
    g = jnp.stack(gs)

    h_mean = jax.lax.dot_general(
        jnp.ones((S,), jnp.float32), lcv, (((0,), (1,)), ((), ())),
        precision="highest") / S

    vblk = (jnp.eye(S, dtype=jnp.float32)[:, None, :]
            * v_ptr[None, :, None]).reshape(S * H, S)

    idxs, lps = pl.pallas_call(
        _decode,
        out_shape=(jax.ShapeDtypeStruct((B, n), jnp.int32),
                   jax.ShapeDtypeStruct((B, n), jnp.float32)),
    )(lcv, lcv.reshape(B, S * E), g, mask, h_mean, low_init_w.reshape(1, -1),
      W_ctx, b_ctx.reshape(1, -1), W_vw, b_vw.reshape(1, -1), Wq,
      bq.reshape(1, -1), Wref, bref.reshape(1, -1), vblk)
    return idxs, lps


# speedup vs baseline: 2.3317x; 1.0439x over previous
import jax
import jax.numpy as jnp
from jax.experimental import pallas as pl

_C = 10.0
_NEG = -1e8
_EPS = 1e-10


def _decode(lcv_ref, lcvw_ref, g_ref, mask_ref, hmean_ref, liw_ref, wctx_ref,
            bctx_ref, wvw_ref, bvw_ref, wq_ref, bq_ref, wref_ref, bref_ref,
            vblk_ref, idx_out_ref, lp_out_ref):
    lcv = lcv_ref[...]
    B, S, E = lcv.shape
    n = S - 1
    H = wq_ref.shape[1]

    h_bar = jnp.dot(hmean_ref[...], wctx_ref[...]) + bctx_ref[...]
    h_rest0 = jnp.dot(liw_ref[...], wvw_ref[...]) + bvw_ref[...]
    query0 = h_bar + h_rest0
    lcvw = lcvw_ref[...]
    wref = wref_ref[...]
    bref = bref_ref[...]
    refw = jnp.concatenate(
        [jnp.dot(lcvw[:, s * E:(s + 1) * E], wref) + bref for s in range(S)],
        axis=1)
    vblk = vblk_ref[...]

    col = jax.lax.broadcasted_iota(jnp.int32, (B, S), 1)
    col3 = jax.lax.broadcasted_iota(jnp.int32, (B, S, 1), 1)
    lane = jax.lax.broadcasted_iota(jnp.int32, (B, n), 1)
    msk0 = jnp.where(col == 0, 1.0, mask_ref[...])

    wq = wq_ref[...]
    bq = bq_ref[...]
    wvw = wvw_ref[...]
    bvw = bvw_ref[...]

    def body(i, carry):
        query, msk, init_h, idx_all, lp_all = carry
        q = jnp.dot(query, wq) + bq
        qw = jnp.concatenate([q] * S, axis=1)
        t = jnp.tanh(refw + qw)
        s = jax.lax.dot_general(t, vblk, (((1,), (0,)), ((), ())))
        u = _C * jnp.tanh(s)
        u = jnp.where(msk > 0, _NEG, u)
        m = jnp.max(u, axis=1, keepdims=True)
        e = jnp.exp(u - m)
        z = jnp.sum(e, axis=1, keepdims=True)
        p = e / z
        logits = jnp.log(p + _EPS)
        score = g_ref[i] + logits
        maxv = jnp.max(score, axis=1, keepdims=True)
        cand = jnp.where(score == maxv, col, S)
        idxc = jnp.min(cand, axis=1, keepdims=True)
        ohb = col == idxc
        lpc = jnp.max(jnp.where(ohb, logits, -jnp.inf), axis=1, keepdims=True)
        msk = jnp.where(ohb, 1.0, msk)
        h = jnp.sum(jnp.where(col3 == idxc[:, :, None], lcv, 0.0), axis=1)
        init_h = jnp.where(i == 0, h, init_h)
        cc = jnp.concatenate([init_h, h], axis=-1)
        query = h_bar + (jnp.dot(cc, wvw) + bvw)
        idx_all = jnp.where(lane == i, idxc, idx_all)
        lp_all = jnp.where(lane == i, lpc, lp_all)
        return query, msk, init_h, idx_all, lp_all

    init = (query0, msk0, jnp.zeros_like(h_bar),
            jnp.zeros((B, n), jnp.int32), jnp.zeros((B, n), jnp.float32))
    _, _, _, idx_all, lp_all = jax.lax.fori_loop(0, n, body, init)
    idx_out_ref[...] = idx_all
    lp_out_ref[...] = lp_all


def kernel(low_context_vector, original_node, mask, low_init_w, W_ctx, b_ctx,
           W_vw, b_vw, Wq, bq, Wref, bref, v_ptr):
    del original_node
    lcv = low_context_vector
    B, S, E = lcv.shape
    H = Wq.shape[1]
    n = S - 1

    k = jax.random.key(42)
    gs = []
    for _ in range(n):
        k, sub = jax.random.split(k)
        gs.append(jax.random.gumbel(sub, (B, S), jnp.float32))
    g = jnp.stack(gs)

    h_mean = jax.lax.dot_general(
        jnp.ones((S,), jnp.float32), lcv, (((0,), (1,)), ((), ())),
        precision="highest") / S

    vblk = (jnp.eye(S, dtype=jnp.float32)[:, None, :]
            * v_ptr[None, :, None]).reshape(S * H, S)

    idxs, lps = pl.pallas_call(
        _decode,
        out_shape=(jax.ShapeDtypeStruct((B, n), jnp.int32),
                   jax.ShapeDtypeStruct((B, n), jnp.float32)),
    )(lcv, lcv.reshape(B, S * E), g, mask, h_mean, low_init_w.reshape(1, -1),
      W_ctx, b_ctx.reshape(1, -1), W_vw, b_vw.reshape(1, -1), Wq,
      bq.reshape(1, -1), Wref, bref.reshape(1, -1), vblk)
    return idxs, lps
